# baseline (speedup 1.0000x reference)
import functools

import jax
import jax.numpy as jnp
from jax import lax
from jax.experimental import pallas as pl
from jax.experimental.pallas import tpu as pltpu
from jax.experimental.pallas import tpu_sc as plsc

N = 10000
E = 160000
D = 256
H = 256

NT = 16
NC = 2
EPT = E // NT
CW = 128
CH = -(-EPT // CW)
EPT_PAD = CH * CW
PAD_ROW = N + 100
NR = N // 2
ACC_ROWS = 5120
SENT = NR + 56
ZCH = ACC_ROWS // NT
FL_A = 320
FL_B = NR - 15 * FL_A


@functools.cache
def _make_sc_agg(count_deg: bool):
    mesh = plsc.VectorSubcoreMesh(core_axis_name="c", subcore_axis_name="s")
    out_type = [jax.ShapeDtypeStruct((2, NC, N, 128), jnp.float32)]
    scratch = [
        pltpu.VMEM((CH, CW), jnp.int32),
        pltpu.VMEM((CH, CW), jnp.int32),
        pltpu.VMEM((CH, CW), jnp.int32),
        pltpu.VMEM((CW, 128), jnp.float32),
        pltpu.VMEM((CW, 128), jnp.float32),
        pltpu.VMEM_SHARED((ACC_ROWS, 128), jnp.float32),
        pltpu.SemaphoreType.DMA,
    ]
    if count_deg:
        out_type.append(jax.ShapeDtypeStruct((2, N, 128), jnp.float32))

    def body(xflat, gidx, sidx, zeros, ones_in, *rest):
        if count_deg:
            s_out, deg_out, gv, sv, svp, rows_a, rows_b, acc, sem = rest
        else:
            s_out, gv, sv, svp, rows_a, rows_b, acc, sem = rest
        c = lax.axis_index("c")
        t = lax.axis_index("s")

        def localize(p):
            lo = p * NR

            def loc_body(i, carry):
                j = i // 8
                k = (i % 8) * 16
                v = sv[j, pl.ds(k, 16)] - lo
                ok = (v >= 0) & (v < NR)
                svp[j, pl.ds(k, 16)] = jnp.where(ok, v, SENT)
                return carry
            lax.fori_loop(0, CH * 8, loc_body, 0)

        def zero_acc():
            pltpu.sync_copy(zeros, acc.at[pl.ds(t * ZCH, ZCH)])

        def flush(dst, p):
            base = p * NR

            @pl.when(t < 15)
            def _():
                pltpu.sync_copy(acc.at[pl.ds(t * FL_A, FL_A)],
                                dst.at[pl.ds(base + t * FL_A, FL_A)])

            @pl.when(t == 15)
            def _():
                pltpu.sync_copy(acc.at[pl.ds(15 * FL_A, FL_B)],
                                dst.at[pl.ds(base + 15 * FL_A, FL_B)])

        if count_deg:
            pltpu.sync_copy(sidx.at[t, c], sv)
            pltpu.sync_copy(ones_in, rows_a)
            for p in range(2):
                localize(p)
                zero_acc()
                plsc.subcore_barrier()

                def cnt_body(j, carry):
                    pltpu.sync_copy(rows_a, acc.at[svp.at[j]], add=True)
                    return carry
                lax.fori_loop(0, CH, cnt_body, 0)
                plsc.subcore_barrier()
                flush(deg_out.at[c], p)
                plsc.subcore_barrier()

        for d in range(2):
            pltpu.sync_copy(gidx.at[t, d], gv)
            pltpu.sync_copy(sidx.at[t, d], sv)

            def adj_body(i, carry):
                j = i // 8
                k = (i % 8) * 16
                v = gv[j, pl.ds(k, 16)]
                gv[j, pl.ds(k, 16)] = v * 2 + c
                return carry
            lax.fori_loop(0, CH * 8, adj_body, 0)

            for p in range(2):
                localize(p)
                zero_acc()
                plsc.subcore_barrier()

                pltpu.async_copy(xflat.at[gv.at[0]], rows_a, sem)

                def chunk_body(i, carry):
                    j0 = 2 * i
                    pltpu.async_copy(xflat.at[gv.at[j0 + 1]], rows_b, sem)
                    pltpu.make_async_copy(xflat.at[gv.at[j0]], rows_a,
                                          sem).wait()
                    pltpu.sync_copy(rows_a, acc.at[svp.at[j0]], add=True)

                    @pl.when(j0 + 2 < CH)
                    def _():
                        pltpu.async_copy(xflat.at[gv.at[j0 + 2]], rows_a,
                                         sem)
                    pltpu.make_async_copy(xflat.at[gv.at[j0 + 1]], rows_b,
                                          sem).wait()
                    pltpu.sync_copy(rows_b, acc.at[svp.at[j0 + 1]],
                                    add=True)
                    return carry
                lax.fori_loop(0, CH // 2, chunk_body, 0)
                pltpu.make_async_copy(xflat.at[gv.at[CH - 1]], rows_a,
                                      sem).wait()
                pltpu.sync_copy(rows_a, acc.at[svp.at[CH - 1]], add=True)

                plsc.subcore_barrier()
                flush(s_out.at[d, c], p)
                plsc.subcore_barrier()

    return pl.kernel(body, out_type=tuple(out_type) if count_deg
                     else out_type[0], mesh=mesh,
                     scratch_types=tuple(scratch))


def _dense_body(x, s, deg, lw, rw, rb, g, b):
    s1 = jnp.concatenate([s[0, 0], s[0, 1]], axis=-1)
    s2 = jnp.concatenate([s[1, 0], s[1, 1]], axis=-1)
    invdt = 1.0 / jnp.maximum(deg[0, :, 0], 1.0)
    invds = 1.0 / jnp.maximum(deg[1, :, 0], 1.0)
    agg = invdt[:, None] * s1 + invds[:, None] * s2
    dn = (((1,), (1,)), ((), ()))
    pre = (lax.dot_general(x, rw[...], dn, preferred_element_type=jnp.float32)
           + lax.dot_general(agg, lw[...], dn,
                             preferred_element_type=jnp.float32)
           + rb[...][None, :])
    h = jnp.maximum(pre, 0.0)
    mu = jnp.mean(h, axis=-1, keepdims=True)
    var = jnp.mean((h - mu) ** 2, axis=-1, keepdims=True)
    h = (h - mu) * lax.rsqrt(var + 1e-5) * g[...][None, :] + b[...][None, :]
    return h


def _dense_mid_kernel(x, s, deg, lw, rw, rb, g, b, out):
    out[...] = _dense_body(x[...], s, deg, lw, rw, rb, g, b)


def _dense_last_kernel(x, s, deg, lw, rw, rb, g, b, x0, h1, fw, fb, out):
    h = _dense_body(x[...], s, deg, lw, rw, rb, g, b)
    w = fw[...]
    dn = (((1,), (1,)), ((), ()))
    f = (lax.dot_general(x0[...], w[:, 0:256], dn,
                         preferred_element_type=jnp.float32)
         + lax.dot_general(h1[...], w[:, 256:512], dn,
                           preferred_element_type=jnp.float32)
         + lax.dot_general(x[...], w[:, 512:768], dn,
                           preferred_element_type=jnp.float32)
         + lax.dot_general(h, w[:, 768:1024], dn,
                           preferred_element_type=jnp.float32)
         + fb[...][None, :])
    out[...] = f


_R = 1000


def _row_specs():
    return [
        pl.BlockSpec((_R, D), lambda i: (i, 0)),
        pl.BlockSpec((2, NC, _R, 128), lambda i: (0, 0, i, 0)),
        pl.BlockSpec((2, _R, 128), lambda i: (0, i, 0)),
        pl.BlockSpec((H, D), lambda i: (0, 0)),
        pl.BlockSpec((H, D), lambda i: (0, 0)),
        pl.BlockSpec((H,), lambda i: (0,)),
        pl.BlockSpec((H,), lambda i: (0,)),
        pl.BlockSpec((H,), lambda i: (0,)),
    ]


_dense_mid = pl.pallas_call(
    _dense_mid_kernel,
    grid=(N // _R,),
    in_specs=_row_specs(),
    out_specs=pl.BlockSpec((_R, D), lambda i: (i, 0)),
    out_shape=jax.ShapeDtypeStruct((N, D), jnp.float32),
)

_dense_last = pl.pallas_call(
    _dense_last_kernel,
    grid=(N // _R,),
    in_specs=_row_specs() + [
        pl.BlockSpec((_R, D), lambda i: (i, 0)),
        pl.BlockSpec((_R, D), lambda i: (i, 0)),
        pl.BlockSpec((H, D + 3 * H), lambda i: (0, 0)),
        pl.BlockSpec((H,), lambda i: (0,)),
    ],
    out_specs=pl.BlockSpec((_R, H), lambda i: (i, 0)),
    out_shape=jax.ShapeDtypeStruct((N, H), jnp.float32),
)


def _prep_indices(edge_index):
    src_n = edge_index[0]
    tgt_n = edge_index[1]

    def split_pad(a, fill):
        a = a.reshape(NT, EPT)
        a = jnp.pad(a, ((0, 0), (0, EPT_PAD - EPT)), constant_values=fill)
        return a.reshape(NT, CH, CW)

    g0 = split_pad(src_n, 0)
    g1 = split_pad(tgt_n, 0)
    s0 = split_pad(tgt_n, PAD_ROW)
    s1 = split_pad(src_n, PAD_ROW)
    gidx = jnp.stack([g0, g1], axis=1)
    sidx = jnp.stack([s0, s1], axis=1)
    return gidx, sidx


def kernel(x, edge_index, lin1_0, lin1_1, lin1_2, root_w_0, root_w_1,
           root_w_2, root_b_0, root_b_1, root_b_2, ln_g_0, ln_g_1, ln_g_2,
           ln_b_0, ln_b_1, ln_b_2, final_w, final_b):
    gidx, sidx = _prep_indices(edge_index)
    zeros = jnp.zeros((ZCH, 128), jnp.float32)
    ones_in = jnp.ones((CW, 128), jnp.float32)

    s_sum, deg = _make_sc_agg(True)(x.reshape(2 * N, 128), gidx, sidx,
                                    zeros, ones_in)
    h1 = _dense_mid(x, s_sum, deg, lin1_0, root_w_0, root_b_0,
                    ln_g_0, ln_b_0)
    s_sum = _make_sc_agg(False)(h1.reshape(2 * N, 128), gidx, sidx, zeros,
                                ones_in)
    h2 = _dense_mid(h1, s_sum, deg, lin1_1, root_w_1, root_b_1,
                    ln_g_1, ln_b_1)
    s_sum = _make_sc_agg(False)(h2.reshape(2 * N, 128), gidx, sidx, zeros,
                                ones_in)
    f = _dense_last(h2, s_sum, deg, lin1_2, root_w_2, root_b_2,
                    ln_g_2, ln_b_2, x, h1, final_w, final_b)
    return f


# speedup vs baseline: 1.1990x; 1.1990x over previous
import functools

import jax
import jax.numpy as jnp
from jax import lax
from jax.experimental import pallas as pl
from jax.experimental.pallas import tpu as pltpu
from jax.experimental.pallas import tpu_sc as plsc

N = 10000
E = 160000
D = 256
H = 256

NT = 16
NC = 2
EPT = E // NT
CW = 64
CH = -(-EPT // CW)
EPT_PAD = CH * CW
PAD_ROW = N + 100
NR = N // 2
ACC_ROWS = 5120
SENT = NR + 56
ZCH = ACC_ROWS // NT
FL_A = 320
FL_B = NR - 15 * FL_A


@functools.cache
def _make_sc_agg(count_deg: bool):
    mesh = plsc.VectorSubcoreMesh(core_axis_name="c", subcore_axis_name="s")
    out_type = [jax.ShapeDtypeStruct((2, NC, N, 128), jnp.float32)]
    scratch = [
        pltpu.VMEM((CH, CW), jnp.int32),
        pltpu.VMEM((CH, CW), jnp.int32),
        pltpu.VMEM((CH, CW), jnp.int32),
        pltpu.VMEM((CW, 128), jnp.float32),
        pltpu.VMEM((CW, 128), jnp.float32),
        pltpu.VMEM((CW, 128), jnp.float32),
        pltpu.VMEM_SHARED((ACC_ROWS, 128), jnp.float32),
        pltpu.SemaphoreType.DMA,
        pltpu.SemaphoreType.DMA,
    ]
    if count_deg:
        out_type.append(jax.ShapeDtypeStruct((2, N, 128), jnp.float32))

    def body(xflat, gidx, sidx, zeros, ones_in, *rest):
        if count_deg:
            (s_out, deg_out, gv, sv, svp, rows_a, rows_b, rows_c, acc,
             sem_g, sem_s) = rest
        else:
            (s_out, gv, sv, svp, rows_a, rows_b, rows_c, acc,
             sem_g, sem_s) = rest
        c = lax.axis_index("c")
        t = lax.axis_index("s")

        def localize(p):
            lo = p * NR

            def loc_body(i, carry):
                j = i // 4
                k = (i % 4) * 16
                v = sv[j, pl.ds(k, 16)] - lo
                ok = (v >= 0) & (v < NR)
                svp[j, pl.ds(k, 16)] = jnp.where(ok, v, SENT)
                return carry
            lax.fori_loop(0, CH * 4, loc_body, 0)

        def zero_acc():
            pltpu.sync_copy(zeros, acc.at[pl.ds(t * ZCH, ZCH)])

        def flush(dst, p):
            base = p * NR

            @pl.when(t < 15)
            def _():
                pltpu.sync_copy(acc.at[pl.ds(t * FL_A, FL_A)],
                                dst.at[pl.ds(base + t * FL_A, FL_A)])

            @pl.when(t == 15)
            def _():
                pltpu.sync_copy(acc.at[pl.ds(15 * FL_A, FL_B)],
                                dst.at[pl.ds(base + 15 * FL_A, FL_B)])

        if count_deg:
            pltpu.sync_copy(sidx.at[t, c], sv)
            pltpu.sync_copy(ones_in, rows_a)
            for p in range(2):
                localize(p)
                zero_acc()
                plsc.subcore_barrier()

                def cnt_body(j, carry):
                    pltpu.sync_copy(rows_a, acc.at[svp.at[j]], add=True)
                    return carry
                lax.fori_loop(0, CH, cnt_body, 0)
                plsc.subcore_barrier()
                flush(deg_out.at[c], p)
                plsc.subcore_barrier()

        for d in range(2):
            pltpu.sync_copy(gidx.at[t, d], gv)
            pltpu.sync_copy(sidx.at[t, d], sv)

            def adj_body(i, carry):
                j = i // 4
                k = (i % 4) * 16
                v = gv[j, pl.ds(k, 16)]
                gv[j, pl.ds(k, 16)] = v * 2 + c
                return carry
            lax.fori_loop(0, CH * 4, adj_body, 0)

            for p in range(2):
                localize(p)
                zero_acc()
                plsc.subcore_barrier()

                def step(j, buf, nxt):
                    pltpu.make_async_copy(xflat.at[gv.at[j]], buf,
                                          sem_g).wait()
                    pltpu.async_copy(buf, acc.at[svp.at[j]], sem_s,
                                     add=True)

                    @pl.when(j + 2 < CH)
                    def _():
                        @pl.when(j > 0)
                        def _():
                            pltpu.make_async_copy(
                                buf, acc.at[svp.at[j]], sem_s).wait()
                        pltpu.async_copy(xflat.at[gv.at[j + 2]], nxt,
                                         sem_g)

                pltpu.async_copy(xflat.at[gv.at[0]], rows_a, sem_g)
                pltpu.async_copy(xflat.at[gv.at[1]], rows_b, sem_g)

                def chunk_body(i, carry):
                    j0 = 3 * i
                    step(j0, rows_a, rows_c)
                    step(j0 + 1, rows_b, rows_a)
                    step(j0 + 2, rows_c, rows_b)
                    return carry
                lax.fori_loop(0, CH // 3, chunk_body, 0)
                pltpu.make_async_copy(xflat.at[gv.at[CH - 1]], rows_a,
                                      sem_g).wait()
                pltpu.async_copy(rows_a, acc.at[svp.at[CH - 1]], sem_s,
                                 add=True)
                for _ in range(3):
                    pltpu.make_async_copy(rows_a, acc.at[svp.at[CH - 1]],
                                          sem_s).wait()

                plsc.subcore_barrier()
                flush(s_out.at[d, c], p)
                plsc.subcore_barrier()

    return pl.kernel(body, out_type=tuple(out_type) if count_deg
                     else out_type[0], mesh=mesh,
                     scratch_types=tuple(scratch))


def _dense_body(x, s, deg, lw, rw, rb, g, b):
    s1 = jnp.concatenate([s[0, 0], s[0, 1]], axis=-1)
    s2 = jnp.concatenate([s[1, 0], s[1, 1]], axis=-1)
    invdt = 1.0 / jnp.maximum(deg[0, :, 0], 1.0)
    invds = 1.0 / jnp.maximum(deg[1, :, 0], 1.0)
    agg = invdt[:, None] * s1 + invds[:, None] * s2
    dn = (((1,), (1,)), ((), ()))
    pre = (lax.dot_general(x, rw[...], dn, preferred_element_type=jnp.float32)
           + lax.dot_general(agg, lw[...], dn,
                             preferred_element_type=jnp.float32)
           + rb[...][None, :])
    h = jnp.maximum(pre, 0.0)
    mu = jnp.mean(h, axis=-1, keepdims=True)
    var = jnp.mean((h - mu) ** 2, axis=-1, keepdims=True)
    h = (h - mu) * lax.rsqrt(var + 1e-5) * g[...][None, :] + b[...][None, :]
    return h


def _dense_mid_kernel(x, s, deg, lw, rw, rb, g, b, out):
    out[...] = _dense_body(x[...], s, deg, lw, rw, rb, g, b)


def _dense_last_kernel(x, s, deg, lw, rw, rb, g, b, x0, h1, fw, fb, out):
    h = _dense_body(x[...], s, deg, lw, rw, rb, g, b)
    w = fw[...]
    dn = (((1,), (1,)), ((), ()))
    f = (lax.dot_general(x0[...], w[:, 0:256], dn,
                         preferred_element_type=jnp.float32)
         + lax.dot_general(h1[...], w[:, 256:512], dn,
                           preferred_element_type=jnp.float32)
         + lax.dot_general(x[...], w[:, 512:768], dn,
                           preferred_element_type=jnp.float32)
         + lax.dot_general(h, w[:, 768:1024], dn,
                           preferred_element_type=jnp.float32)
         + fb[...][None, :])
    out[...] = f


_R = 1000


def _row_specs():
    return [
        pl.BlockSpec((_R, D), lambda i: (i, 0)),
        pl.BlockSpec((2, NC, _R, 128), lambda i: (0, 0, i, 0)),
        pl.BlockSpec((2, _R, 128), lambda i: (0, i, 0)),
        pl.BlockSpec((H, D), lambda i: (0, 0)),
        pl.BlockSpec((H, D), lambda i: (0, 0)),
        pl.BlockSpec((H,), lambda i: (0,)),
        pl.BlockSpec((H,), lambda i: (0,)),
        pl.BlockSpec((H,), lambda i: (0,)),
    ]


_dense_mid = pl.pallas_call(
    _dense_mid_kernel,
    grid=(N // _R,),
    in_specs=_row_specs(),
    out_specs=pl.BlockSpec((_R, D), lambda i: (i, 0)),
    out_shape=jax.ShapeDtypeStruct((N, D), jnp.float32),
)

_dense_last = pl.pallas_call(
    _dense_last_kernel,
    grid=(N // _R,),
    in_specs=_row_specs() + [
        pl.BlockSpec((_R, D), lambda i: (i, 0)),
        pl.BlockSpec((_R, D), lambda i: (i, 0)),
        pl.BlockSpec((H, D + 3 * H), lambda i: (0, 0)),
        pl.BlockSpec((H,), lambda i: (0,)),
    ],
    out_specs=pl.BlockSpec((_R, H), lambda i: (i, 0)),
    out_shape=jax.ShapeDtypeStruct((N, H), jnp.float32),
)


def _prep_indices(edge_index):
    src_n = edge_index[0]
    tgt_n = edge_index[1]

    def split_pad(a, fill):
        a = a.reshape(NT, EPT)
        a = jnp.pad(a, ((0, 0), (0, EPT_PAD - EPT)), constant_values=fill)
        return a.reshape(NT, CH, CW)

    g0 = split_pad(src_n, 0)
    g1 = split_pad(tgt_n, 0)
    s0 = split_pad(tgt_n, PAD_ROW)
    s1 = split_pad(src_n, PAD_ROW)
    gidx = jnp.stack([g0, g1], axis=1)
    sidx = jnp.stack([s0, s1], axis=1)
    return gidx, sidx


def kernel(x, edge_index, lin1_0, lin1_1, lin1_2, root_w_0, root_w_1,
           root_w_2, root_b_0, root_b_1, root_b_2, ln_g_0, ln_g_1, ln_g_2,
           ln_b_0, ln_b_1, ln_b_2, final_w, final_b):
    gidx, sidx = _prep_indices(edge_index)
    zeros = jnp.zeros((ZCH, 128), jnp.float32)
    ones_in = jnp.ones((CW, 128), jnp.float32)

    s_sum, deg = _make_sc_agg(True)(x.reshape(2 * N, 128), gidx, sidx,
                                    zeros, ones_in)
    h1 = _dense_mid(x, s_sum, deg, lin1_0, root_w_0, root_b_0,
                    ln_g_0, ln_b_0)
    s_sum = _make_sc_agg(False)(h1.reshape(2 * N, 128), gidx, sidx, zeros,
                                ones_in)
    h2 = _dense_mid(h1, s_sum, deg, lin1_1, root_w_1, root_b_1,
                    ln_g_1, ln_b_1)
    s_sum = _make_sc_agg(False)(h2.reshape(2 * N, 128), gidx, sidx, zeros,
                                ones_in)
    f = _dense_last(h2, s_sum, deg, lin1_2, root_w_2, root_b_2,
                    ln_g_2, ln_b_2, x, h1, final_w, final_b)
    return f
